# baseline (speedup 1.0000x reference)
import functools

import jax
import jax.numpy as jnp
from jax import lax
from jax.experimental import pallas as pl
from jax.experimental.pallas import tpu as pltpu
from jax.experimental.pallas import tpu_sc as plsc

_H = 8


@functools.cache
def _build(B, V, D):
    info = plsc.get_sparse_core_info()
    nw = info.num_cores * info.num_subcores
    b_per_w = B // nw
    n_pair = b_per_w // (2 * _H)
    n_half = 2 * n_pair
    assert (n_half - 4) % 6 == 0
    n_iter = (n_half - 4) // 6
    mesh = plsc.VectorSubcoreMesh(core_axis_name="c", subcore_axis_name="s")

    @functools.partial(
        pl.kernel,
        mesh=mesh,
        out_type=jax.ShapeDtypeStruct((D, B), jnp.float32),
        compiler_params=pltpu.CompilerParams(needs_layout_passes=False),
        scratch_types=[
            pltpu.VMEM((1, n_pair, 2 * _H), jnp.int32),
            pltpu.VMEM((D, _H * 128), jnp.float32),
            pltpu.VMEM((D, _H * 128), jnp.float32),
            pltpu.VMEM((D, _H * 128), jnp.float32),
            pltpu.VMEM((D, b_per_w), jnp.float32),
            pltpu.SemaphoreType.DMA,
            pltpu.SemaphoreType.DMA,
            pltpu.SemaphoreType.DMA,
        ],
    )
    def gather_kernel(
        table_hbm, idx_hbm, out_hbm, idx_v, b0, b1, b2, cols_v,
        s0, s1, s2,
    ):
        bufs = (b0, b1, b2)
        sems = (s0, s1, s2)
        wid = lax.axis_index("s") * info.num_cores + lax.axis_index("c")
        base = pl.multiple_of(wid * b_per_w, 128)
        pltpu.sync_copy(idx_hbm.at[pl.ds(wid, 1)], idx_v)
        lane16 = lax.iota(jnp.int32, 16)
        half16 = (lane16 & 7) * 128

        def fire(p, hi, slot):
            v = idx_v[0, p]
            for u in range(_H):
                a = pl.multiple_of((v[u + _H * hi] >> 7) << 7, 128)
                pltpu.async_copy(
                    table_hbm.at[:, pl.ds(a, 128)],
                    bufs[slot].at[:, pl.ds(u * 128, 128)],
                    sems[slot],
                )

        def drain(slot):
            for _ in range(_H):
                pltpu.make_async_copy(
                    table_hbm.at[:, pl.ds(0, 128)],
                    bufs[slot].at[:, pl.ds(0, 128)],
                    sems[slot],
                ).wait()

        def extract(p, slot_a, slot_b):
            v = idx_v[0, p]
            pos = half16 + (v & 127)
            zero16 = lane16 * 0
            sel = lane16 < 8
            for d in range(D):
                ga = plsc.load_gather(bufs[slot_a], [zero16 + d, pos])
                gb = plsc.load_gather(bufs[slot_b], [zero16 + d, pos])
                cols_v[d, pl.ds(p * 16, 16)] = jnp.where(sel, ga, gb)

        fire(0, 0, 0)
        fire(0, 1, 1)

        def body(k):
            h = 6 * k
            p = 3 * k
            fire(p + 1, 0, 2)
            drain(0)
            drain(1)
            extract(p, 0, 1)
            fire(p + 1, 1, 0)
            fire(p + 2, 0, 1)
            drain(2)
            drain(0)
            extract(p + 1, 2, 0)
            fire(p + 2, 1, 2)
            fire(p + 3, 0, 0)
            drain(1)
            drain(2)
            extract(p + 2, 1, 2)
            fire(p + 3, 1, 1)

        pl.loop(0, n_iter)(body)

        fire(n_pair - 1, 0, 2)
        drain(0)
        drain(1)
        extract(n_pair - 2, 0, 1)
        fire(n_pair - 1, 1, 0)
        drain(2)
        drain(0)
        extract(n_pair - 1, 2, 0)

        pltpu.sync_copy(cols_v, out_hbm.at[:, pl.ds(base, b_per_w)])

    return gather_kernel, nw, n_pair


def kernel(item_id, table):
    B, = item_id.shape
    V, D = table.shape
    gather_kernel, nw, n_pair = _build(B, V, D)
    idx = item_id.astype(jnp.int32).reshape(nw, n_pair, 2 * _H)
    out_t = gather_kernel(table.T, idx)
    return out_t.T


# speedup vs baseline: 1.0491x; 1.0491x over previous
import functools

import jax
import jax.numpy as jnp
from jax import lax
from jax.experimental import pallas as pl
from jax.experimental.pallas import tpu as pltpu
from jax.experimental.pallas import tpu_sc as plsc

_G = 16


@functools.cache
def _build(B, V, D):
    info = plsc.get_sparse_core_info()
    nw = info.num_cores * info.num_subcores
    b_per_w = B // nw
    n_grp = b_per_w // _G
    mesh = plsc.VectorSubcoreMesh(core_axis_name="c", subcore_axis_name="s")

    @functools.partial(
        pl.kernel,
        mesh=mesh,
        out_type=jax.ShapeDtypeStruct((D, B), jnp.float32),
        compiler_params=pltpu.CompilerParams(needs_layout_passes=False),
        scratch_types=[
            pltpu.VMEM((1, n_grp, _G), jnp.int32),
            pltpu.VMEM((D, _G * 128), jnp.float32),
            pltpu.VMEM((D, b_per_w), jnp.float32),
            pltpu.SemaphoreType.DMA,
        ],
    )
    def gather_kernel(table_hbm, idx_hbm, out_hbm, idx_v, buf_v, cols_v, sem):
        wid = lax.axis_index("s") * info.num_cores + lax.axis_index("c")
        base = pl.multiple_of(wid * b_per_w, 128)
        pltpu.sync_copy(idx_hbm.at[pl.ds(wid, 1)], idx_v)
        lane16 = lax.iota(jnp.int32, _G)

        def group(g):
            ivec = idx_v[0, g]
            copies = []
            for u in range(_G):
                a = pl.multiple_of((ivec[u] >> 7) << 7, 128)
                copies.append(
                    pltpu.async_copy(
                        table_hbm.at[:, pl.ds(a, 128)],
                        buf_v.at[:, pl.ds(u * 128, 128)],
                        sem,
                    )
                )
            for c in copies:
                c.wait()
            pos = lane16 * 128 + (ivec & 127)
            zero16 = lane16 * 0
            for d in range(D):
                cols_v[d, pl.ds(g * _G, _G)] = plsc.load_gather(
                    buf_v, [zero16 + d, pos]
                )

        pl.loop(0, n_grp)(group)
        pltpu.sync_copy(cols_v, out_hbm.at[:, pl.ds(base, b_per_w)])

    return gather_kernel, nw, n_grp


def kernel(item_id, table):
    B, = item_id.shape
    V, D = table.shape
    gather_kernel, nw, n_grp = _build(B, V, D)
    idx = item_id.astype(jnp.int32).reshape(nw, n_grp, _G)
    out_t = gather_kernel(table.T, idx)
    return out_t.T
